# Initial kernel scaffold; baseline (speedup 1.0000x reference)
import jax
import jax.numpy as jnp
from jax.experimental import pallas as pl


def kernel(node_features, codebook):
    raise NotImplementedError("write your pallas kernel here")


# speedup vs baseline: 8.1512x; 8.1512x over previous
import functools

import jax
import jax.numpy as jnp
from jax import lax
from jax.experimental import pallas as pl
from jax.experimental.pallas import tpu as pltpu
from jax.experimental.pallas import tpu_sc as plsc

N_NODES_IN = 100000
N_FEATS = 9
CB_ROWS = 334
MODEL_DIM = 128

L = 16
NUM_CORES = 2
NUM_SUBCORES = 16
NUM_WORKERS = NUM_CORES * NUM_SUBCORES

PER_WORKER = 3128
N_PAD = NUM_WORKERS * PER_WORKER
CHUNK = 136
NUM_CHUNKS = PER_WORKER // CHUNK

_mesh = plsc.VectorSubcoreMesh(core_axis_name="c", subcore_axis_name="s")


@functools.partial(
    pl.kernel,
    mesh=_mesh,
    compiler_params=pltpu.CompilerParams(needs_layout_passes=False),
    out_type=jax.ShapeDtypeStruct((N_PAD * MODEL_DIM,), jnp.float32),
    scratch_types=[
        pltpu.VMEM((CB_ROWS * MODEL_DIM,), jnp.float32),
        pltpu.VMEM((CHUNK * N_FEATS + L,), jnp.int32),
        pltpu.VMEM((CHUNK * MODEL_DIM,), jnp.float32),
    ],
)
def _embed_sum(idx_hbm, cb_hbm, out_hbm, cb_v, idx_v, out_v):
    wid = lax.axis_index("s") * NUM_CORES + lax.axis_index("c")
    base = wid * PER_WORKER

    pltpu.sync_copy(cb_hbm, cb_v)

    cols = [jnp.arange(L, dtype=jnp.int32) + (db * L) for db in range(MODEL_DIM // L)]

    def chunk_body(c, carry):
        cbase = base + c * CHUNK
        pltpu.sync_copy(
            idx_hbm.at[pl.ds(cbase * N_FEATS, CHUNK * N_FEATS)],
            idx_v.at[pl.ds(0, CHUNK * N_FEATS)],
        )

        def node_body(n, carry2):
            idx_vec = idx_v[pl.ds(n * N_FEATS, L)]
            rows = [
                jnp.full((L,), idx_vec[j], jnp.int32) * MODEL_DIM
                for j in range(N_FEATS)
            ]
            for db in range(MODEL_DIM // L):
                col = cols[db]
                g = [
                    plsc.load_gather(cb_v, [rows[j] + col])
                    for j in range(N_FEATS)
                ]
                s01 = g[0] + g[1]
                s23 = g[2] + g[3]
                s45 = g[4] + g[5]
                s67 = g[6] + g[7]
                acc = (s01 + s23) + (s45 + s67) + g[8]
                out_v[pl.ds(n * MODEL_DIM + db * L, L)] = acc
            return carry2

        lax.fori_loop(0, CHUNK, node_body, 0)
        pltpu.sync_copy(out_v, out_hbm.at[pl.ds(cbase * MODEL_DIM, CHUNK * MODEL_DIM)])
        return carry

    lax.fori_loop(0, NUM_CHUNKS, chunk_body, 0)


def kernel(node_features, codebook):
    nf = node_features.astype(jnp.int32)
    nf = jnp.pad(nf, ((0, N_PAD - N_NODES_IN), (0, 0)))
    out = _embed_sum(nf.reshape(-1), codebook.reshape(-1))
    return out.reshape(N_PAD, MODEL_DIM)[:N_NODES_IN]
